# baseline (speedup 1.0000x reference)
import functools

import jax
import jax.numpy as jnp
from jax import lax
from jax.experimental import pallas as pl
from jax.experimental.pallas import tpu as pltpu
from jax.experimental.pallas import tpu_sc as plsc

_NC = 2
_NS = 16
_NW = _NC * _NS


def kernel(x_dict, edge_index, entity_emb, rel_emb):
    del x_dict, edge_index
    n_ent, d_ent = entity_emb.shape
    rows = (n_ent // _NW) // 8 * 8
    tail_base = rows * _NW
    tail = n_ent - tail_base

    mesh = plsc.VectorSubcoreMesh(core_axis_name="c", subcore_axis_name="s")

    @functools.partial(
        pl.kernel,
        mesh=mesh,
        out_type=[
            jax.ShapeDtypeStruct(entity_emb.shape, entity_emb.dtype),
            jax.ShapeDtypeStruct(rel_emb.shape, rel_emb.dtype),
        ],
        scratch_types=[pltpu.SemaphoreType.DMA, pltpu.SemaphoreType.DMA],
    )
    def _sc_copy(ent_hbm, rel_hbm, ent_out, rel_out, sem, rsem):
        wid = lax.axis_index("s") * _NC + lax.axis_index("c")
        base = wid * rows
        cp = pltpu.make_async_copy(
            ent_hbm.at[pl.ds(base, rows)], ent_out.at[pl.ds(base, rows)], sem
        )
        cp.start()

        @pl.when(wid == 0)
        def _():
            rcp = pltpu.make_async_copy(rel_hbm, rel_out, rsem)
            rcp.start()
            if tail:
                tcp = pltpu.make_async_copy(
                    ent_hbm.at[pl.ds(tail_base, tail)],
                    ent_out.at[pl.ds(tail_base, tail)],
                    rsem,
                )
                tcp.start()
                tcp.wait()
            rcp.wait()

        cp.wait()

    return tuple(_sc_copy(entity_emb, rel_emb))


# speedup vs baseline: 41.0439x; 41.0439x over previous
import functools

import jax
import jax.numpy as jnp
from jax import lax
from jax.experimental import pallas as pl
from jax.experimental.pallas import tpu as pltpu
from jax.experimental.pallas import tpu_sc as plsc

_NC = 2
_NS = 16
_NW = _NC * _NS
_CHUNK = 504


def kernel(x_dict, edge_index, entity_emb, rel_emb):
    del x_dict, edge_index
    n_ent, d_ent = entity_emb.shape
    rows = (n_ent // _NW) // _CHUNK * _CHUNK
    nchunks = rows // _CHUNK
    tail_base = rows * _NW
    tail = n_ent - tail_base

    mesh = plsc.VectorSubcoreMesh(core_axis_name="c", subcore_axis_name="s")

    @functools.partial(
        pl.kernel,
        mesh=mesh,
        out_type=[
            jax.ShapeDtypeStruct(entity_emb.shape, entity_emb.dtype),
            jax.ShapeDtypeStruct(rel_emb.shape, rel_emb.dtype),
        ],
        scratch_types=[
            pltpu.VMEM((_CHUNK, d_ent), entity_emb.dtype),
            pltpu.VMEM((_CHUNK, d_ent), entity_emb.dtype),
            pltpu.SemaphoreType.DMA,
            pltpu.SemaphoreType.DMA,
            pltpu.SemaphoreType.DMA,
            pltpu.SemaphoreType.DMA,
            pltpu.SemaphoreType.DMA,
        ],
    )
    def _sc_copy(ent_hbm, rel_hbm, ent_out, rel_out,
                 buf0, buf1, isem0, isem1, osem0, osem1, rsem):
        wid = lax.axis_index("s") * _NC + lax.axis_index("c")
        base = wid * rows
        bufs = (buf0, buf1)
        isems = (isem0, isem1)
        osems = (osem0, osem1)

        @pl.when(wid == 0)
        def _():
            pltpu.make_async_copy(rel_hbm, rel_out, rsem).start()
            if tail:
                pltpu.make_async_copy(
                    ent_hbm.at[pl.ds(tail_base, tail)],
                    ent_out.at[pl.ds(tail_base, tail)],
                    rsem,
                ).start()

        out_cps = [None, None]
        for i in range(nchunks):
            b = i % 2
            if out_cps[b] is not None:
                out_cps[b].wait()
            lo = base + i * _CHUNK
            icp = pltpu.make_async_copy(
                ent_hbm.at[pl.ds(lo, _CHUNK)], bufs[b], isems[b]
            )
            icp.start()
            icp.wait()
            ocp = pltpu.make_async_copy(
                bufs[b], ent_out.at[pl.ds(lo, _CHUNK)], osems[b]
            )
            ocp.start()
            out_cps[b] = ocp
        for cp in out_cps:
            if cp is not None:
                cp.wait()

        @pl.when(wid == 0)
        def _():
            rcp = pltpu.make_async_copy(rel_hbm, rel_out, rsem)
            if tail:
                tcp = pltpu.make_async_copy(
                    ent_hbm.at[pl.ds(tail_base, tail)],
                    ent_out.at[pl.ds(tail_base, tail)],
                    rsem,
                )
                tcp.wait()
            rcp.wait()

    return tuple(_sc_copy(entity_emb, rel_emb))
